# Initial kernel scaffold; baseline (speedup 1.0000x reference)
import jax
import jax.numpy as jnp
from jax.experimental import pallas as pl


def kernel(word, char, pos, lut, tgtwords, rand_idx, xpos_ids):
    raise NotImplementedError("write your pallas kernel here")


# speedup vs baseline: 3.2662x; 3.2662x over previous
import functools

import jax
import jax.numpy as jnp
from jax import lax
from jax.experimental import pallas as pl
from jax.experimental.pallas import tpu as pltpu
from jax.experimental.pallas import tpu_sc as plsc

B = 4096
L = 50
CH = 16
N = B * L
NW = 32
PW = N // NW
HALF = PW // 2
N_TGT = 32
TBL = 64


def _body(word_h, pos_h, rand_h, lut_h, tgt_h, xp_h,
          ow_h, oc_h, om_h,
          xp_v, tgt_v, tbl_v, word_v, pos_v, rand_v, obf_v, mask_v,
          rows_v, sem):
    wid = lax.axis_index("s") * 2 + lax.axis_index("c")
    base = wid * PW

    pltpu.sync_copy(xp_h, xp_v)
    pltpu.sync_copy(tgt_h, tgt_v)

    zeros = jnp.zeros((16,), jnp.int32)
    for i in range(TBL // 16):
        tbl_v[pl.ds(i * 16, 16)] = zeros
    xv = jnp.clip(xp_v[...], 0, TBL - 1)
    plsc.store_scatter(tbl_v, [xv], jnp.ones((16,), jnp.int32))

    pltpu.sync_copy(word_h.at[pl.ds(base, PW)], word_v)
    pltpu.sync_copy(pos_h.at[pl.ds(base, PW)], pos_v)
    pltpu.sync_copy(rand_h.at[pl.ds(base, PW)], rand_v)

    def step(r, _):
        for c in range(8):
            off = r * 128 + c * 16
            pv = pos_v[pl.ds(off, 16)]
            rv = rand_v[pl.ds(off, 16)]
            wv = word_v[pl.ds(off, 16)]
            m = plsc.load_gather(tbl_v, [pv])
            cd = plsc.load_gather(tgt_v, [rv])
            ob = jnp.where(m != 0, cd, wv)
            obf_v[pl.ds(off, 16)] = ob
            mask_v[pl.ds(off, 16)] = m
        return _

    lax.fori_loop(0, PW // 128, step, 0)

    pltpu.sync_copy(obf_v, ow_h.at[pl.ds(base, PW)])
    pltpu.sync_copy(mask_v, om_h.at[pl.ds(base, PW)])

    for h in range(2):
        idx = obf_v.at[pl.ds(h * HALF, HALF)]
        pltpu.async_copy(lut_h.at[idx], rows_v, sem).wait()
        pltpu.sync_copy(rows_v, oc_h.at[pl.ds(base + h * HALF, HALF)])


_mesh = plsc.VectorSubcoreMesh(core_axis_name="c", subcore_axis_name="s")

_sc_call = functools.partial(
    pl.kernel,
    mesh=_mesh,
    compiler_params=pltpu.CompilerParams(
        needs_layout_passes=False, use_tc_tiling_on_sc=False),
    out_type=(
        jax.ShapeDtypeStruct((N,), jnp.int32),
        jax.ShapeDtypeStruct((N, CH), jnp.float32),
        jax.ShapeDtypeStruct((N,), jnp.int32),
    ),
    scratch_types=[
        pltpu.VMEM((16,), jnp.int32),
        pltpu.VMEM((N_TGT,), jnp.int32),
        pltpu.VMEM((TBL,), jnp.int32),
        pltpu.VMEM((PW,), jnp.int32),
        pltpu.VMEM((PW,), jnp.int32),
        pltpu.VMEM((PW,), jnp.int32),
        pltpu.VMEM((PW,), jnp.int32),
        pltpu.VMEM((PW,), jnp.int32),
        pltpu.VMEM((HALF, CH), jnp.float32),
        pltpu.SemaphoreType.DMA,
    ],
)(_body)


@jax.jit
def kernel(word, char, pos, lut, tgtwords, rand_idx, xpos_ids):
    wf = word.reshape(N)
    pf = pos.reshape(N)
    rf = rand_idx.reshape(N)
    pad = jnp.arange(48, 48 + 16 - xpos_ids.shape[0], dtype=jnp.int32)
    xp = jnp.concatenate([xpos_ids, pad])
    ow, oc, om = _sc_call(wf, pf, rf, lut, tgtwords, xp)
    return (
        ow.reshape(B, L),
        oc.reshape(B, L, CH).astype(char.dtype),
        pos,
        (om != 0).reshape(B, L),
    )


# speedup vs baseline: 3.2677x; 1.0005x over previous
import functools

import jax
import jax.numpy as jnp
from jax import lax
from jax.experimental import pallas as pl
from jax.experimental.pallas import tpu as pltpu
from jax.experimental.pallas import tpu_sc as plsc

B = 4096
L = 50
CH = 16
NW = 32
BPW = B // NW
PW = BPW * L
HALFB = BPW // 2
N_TGT = 32
TBL = 64
OFFS = (0, 16, 32, 34)


def _body(word_h, pos_h, rand_h, lut_h, tgt_h, xp_h,
          ow_h, oc_h, om_h,
          xp_v, tgt_v, tbl_v, word_v, pos_v, rand_v, obf_v, mask_v,
          idx_v, rows_v, sem):
    wid = lax.axis_index("s") * 2 + lax.axis_index("c")
    b0 = wid * BPW

    pltpu.sync_copy(xp_h, xp_v)
    pltpu.sync_copy(tgt_h, tgt_v)

    zeros = jnp.zeros((16,), jnp.int32)
    for i in range(TBL // 16):
        tbl_v[pl.ds(i * 16, 16)] = zeros
    xv = jnp.clip(xp_v[...], 0, TBL - 1)
    plsc.store_scatter(tbl_v, [xv], jnp.ones((16,), jnp.int32))

    pltpu.sync_copy(word_h.at[pl.ds(b0, BPW)], word_v)
    pltpu.sync_copy(pos_h.at[pl.ds(b0, BPW)], pos_v)
    pltpu.sync_copy(rand_h.at[pl.ds(b0, BPW)], rand_v)

    def step(r, _):
        for off in OFFS:
            pv = pos_v[r, pl.ds(off, 16)]
            rv = rand_v[r, pl.ds(off, 16)]
            wv = word_v[r, pl.ds(off, 16)]
            m = plsc.load_gather(tbl_v, [pv])
            cd = plsc.load_gather(tgt_v, [rv])
            ob = jnp.where(m != 0, cd, wv)
            obf_v[r, pl.ds(off, 16)] = ob
            mask_v[r, pl.ds(off, 16)] = m
            idx_v[pl.ds(r * L + off, 16)] = ob
        return _

    lax.fori_loop(0, BPW, step, 0)

    pltpu.sync_copy(obf_v, ow_h.at[pl.ds(b0, BPW)])
    pltpu.sync_copy(mask_v, om_h.at[pl.ds(b0, BPW)])

    for h in range(2):
        idx = idx_v.at[pl.ds(h * HALFB * L, HALFB * L)]
        pltpu.async_copy(lut_h.at[idx], rows_v, sem).wait()
        pltpu.sync_copy(rows_v, oc_h.at[pl.ds(wid * PW + h * HALFB * L, HALFB * L)])


_mesh = plsc.VectorSubcoreMesh(core_axis_name="c", subcore_axis_name="s")

_sc_call = functools.partial(
    pl.kernel,
    mesh=_mesh,
    compiler_params=pltpu.CompilerParams(
        needs_layout_passes=False, use_tc_tiling_on_sc=False),
    out_type=(
        jax.ShapeDtypeStruct((B, L), jnp.int32),
        jax.ShapeDtypeStruct((B * L, CH), jnp.float32),
        jax.ShapeDtypeStruct((B, L), jnp.int32),
    ),
    scratch_types=[
        pltpu.VMEM((16,), jnp.int32),
        pltpu.VMEM((N_TGT,), jnp.int32),
        pltpu.VMEM((TBL,), jnp.int32),
        pltpu.VMEM((BPW, L), jnp.int32),
        pltpu.VMEM((BPW, L), jnp.int32),
        pltpu.VMEM((BPW, L), jnp.int32),
        pltpu.VMEM((BPW, L), jnp.int32),
        pltpu.VMEM((BPW, L), jnp.int32),
        pltpu.VMEM((PW,), jnp.int32),
        pltpu.VMEM((HALFB * L, CH), jnp.float32),
        pltpu.SemaphoreType.DMA,
    ],
)(_body)


@jax.jit
def kernel(word, char, pos, lut, tgtwords, rand_idx, xpos_ids):
    pad = jnp.arange(48, 48 + 16 - xpos_ids.shape[0], dtype=jnp.int32)
    xp = jnp.concatenate([xpos_ids, pad])
    ow, oc, om = _sc_call(word, pos, rand_idx, lut, tgtwords, xp)
    return (ow, oc.reshape(B, L, CH).astype(char.dtype), pos, om != 0)


# speedup vs baseline: 5.3327x; 1.6320x over previous
import functools

import jax
import jax.numpy as jnp
from jax import lax
from jax.experimental import pallas as pl
from jax.experimental.pallas import tpu as pltpu
from jax.experimental.pallas import tpu_sc as plsc

B = 4096
L = 50
CH = 16
NW = 32
BPW = B // NW
PW = BPW * L
N_TGT = 32
TBL = 64
LC = 10
NCHUNK = L // LC


def _body(word_h, pos_h, rand_h, lut_h, tgt_h, xp_h,
          ow_h, oc_h, om_h,
          xp_v, tgt_v, tbl_v, word_v, pos_v, rand_v, obf_v, mask_v,
          idx_v, rows_v, trans_v, sem):
    wid = lax.axis_index("s") * 2 + lax.axis_index("c")
    b0 = wid * BPW

    pltpu.sync_copy(xp_h, xp_v)
    pltpu.sync_copy(tgt_h, tgt_v)

    zeros = jnp.zeros((16,), jnp.int32)
    for i in range(TBL // 16):
        tbl_v[pl.ds(i * 16, 16)] = zeros
    xv = jnp.clip(xp_v[...], 0, TBL - 1)
    plsc.store_scatter(tbl_v, [xv], jnp.ones((16,), jnp.int32))

    pltpu.sync_copy(word_h.at[:, pl.ds(b0, BPW)], word_v)
    pltpu.sync_copy(pos_h.at[:, pl.ds(b0, BPW)], pos_v)
    pltpu.sync_copy(rand_h.at[:, pl.ds(b0, BPW)], rand_v)

    def step(l, _):
        for j in range(BPW // 16):
            off = j * 16
            pv = pos_v[l, pl.ds(off, 16)]
            rv = rand_v[l, pl.ds(off, 16)]
            wv = word_v[l, pl.ds(off, 16)]
            m = plsc.load_gather(tbl_v, [pv])
            cd = plsc.load_gather(tgt_v, [rv])
            ob = jnp.where(m != 0, cd, wv)
            obf_v[l, pl.ds(off, 16)] = ob
            mask_v[l, pl.ds(off, 16)] = m
            idx_v[pl.ds(l * BPW + off, 16)] = ob
        return _

    lax.fori_loop(0, L, step, 0)

    pltpu.sync_copy(obf_v, ow_h.at[:, pl.ds(b0, BPW)])
    pltpu.sync_copy(mask_v, om_h.at[:, pl.ds(b0, BPW)])

    iota = lax.iota(jnp.int32, 16)

    for c in range(NCHUNK):
        idx = idx_v.at[pl.ds(c * LC * BPW, LC * BPW)]
        pltpu.async_copy(lut_h.at[idx], rows_v, sem).wait()

        def tstep(lc, _):
            for ch in range(CH):
                cv = jnp.full((16,), ch, jnp.int32)
                for j in range(BPW // 16):
                    ridx = iota + (lc * BPW + j * 16)
                    v = plsc.load_gather(rows_v, [ridx, cv])
                    trans_v[lc, ch, pl.ds(j * 16, 16)] = v
            return _

        lax.fori_loop(0, LC, tstep, 0)
        pltpu.sync_copy(trans_v, oc_h.at[pl.ds(c * LC, LC), :, pl.ds(b0, BPW)])


_mesh = plsc.VectorSubcoreMesh(core_axis_name="c", subcore_axis_name="s")

_sc_call = functools.partial(
    pl.kernel,
    mesh=_mesh,
    compiler_params=pltpu.CompilerParams(
        needs_layout_passes=False, use_tc_tiling_on_sc=False),
    out_type=(
        jax.ShapeDtypeStruct((L, B), jnp.int32),
        jax.ShapeDtypeStruct((L, CH, B), jnp.float32),
        jax.ShapeDtypeStruct((L, B), jnp.int32),
    ),
    scratch_types=[
        pltpu.VMEM((16,), jnp.int32),
        pltpu.VMEM((N_TGT,), jnp.int32),
        pltpu.VMEM((TBL,), jnp.int32),
        pltpu.VMEM((L, BPW), jnp.int32),
        pltpu.VMEM((L, BPW), jnp.int32),
        pltpu.VMEM((L, BPW), jnp.int32),
        pltpu.VMEM((L, BPW), jnp.int32),
        pltpu.VMEM((L, BPW), jnp.int32),
        pltpu.VMEM((PW,), jnp.int32),
        pltpu.VMEM((LC * BPW, CH), jnp.float32),
        pltpu.VMEM((LC, CH, BPW), jnp.float32),
        pltpu.SemaphoreType.DMA,
    ],
)(_body)


@jax.jit
def kernel(word, char, pos, lut, tgtwords, rand_idx, xpos_ids):
    pad = jnp.arange(48, 48 + 16 - xpos_ids.shape[0], dtype=jnp.int32)
    xp = jnp.concatenate([xpos_ids, pad])
    owt, oct_, omt = _sc_call(word.T, pos.T, rand_idx.T, lut, tgtwords, xp)
    return (
        owt.T,
        jnp.transpose(oct_, (2, 0, 1)).astype(char.dtype),
        pos,
        (omt != 0).T,
    )


# speedup vs baseline: 5.8546x; 1.0979x over previous
import functools

import jax
import jax.numpy as jnp
from jax import lax
from jax.experimental import pallas as pl
from jax.experimental.pallas import tpu as pltpu
from jax.experimental.pallas import tpu_sc as plsc

B = 4096
L = 50
CH = 16
NW = 32
BPW = B // NW
PW = BPW * L
N_TGT = 32
TBL = 64
LC = 10
NCHUNK = L // LC
CROWS = LC * BPW


def _body(word_h, pos_h, rand_h, lut_h, tgt_h, xp_h,
          ow_h, oc_h, om_h,
          xp_v, tgt_v, tbl_v, word_v, pos_v, rand_v, obf_v, mask_v,
          idx_v, rows0_v, rows1_v, trans0_v, trans1_v,
          insem, wsem, gsem0, gsem1, osem0, osem1):
    wid = lax.axis_index("s") * 2 + lax.axis_index("c")
    b0 = wid * BPW

    rows = (rows0_v, rows1_v)
    trans = (trans0_v, trans1_v)
    gsem = (gsem0, gsem1)
    osem = (osem0, osem1)

    h_in = [
        pltpu.async_copy(word_h.at[:, pl.ds(b0, BPW)], word_v, insem),
        pltpu.async_copy(pos_h.at[:, pl.ds(b0, BPW)], pos_v, insem),
        pltpu.async_copy(rand_h.at[:, pl.ds(b0, BPW)], rand_v, insem),
    ]
    pltpu.sync_copy(xp_h, xp_v)
    pltpu.sync_copy(tgt_h, tgt_v)

    zeros = jnp.zeros((16,), jnp.int32)
    for i in range(TBL // 16):
        tbl_v[pl.ds(i * 16, 16)] = zeros
    xv = jnp.clip(xp_v[...], 0, TBL - 1)
    plsc.store_scatter(tbl_v, [xv], jnp.ones((16,), jnp.int32))

    for h in h_in:
        h.wait()

    iota = lax.iota(jnp.int32, 16)

    def compute_chunk(c):
        def step(l, _):
            for j in range(BPW // 16):
                off = j * 16
                pv = pos_v[l, pl.ds(off, 16)]
                rv = rand_v[l, pl.ds(off, 16)]
                wv = word_v[l, pl.ds(off, 16)]
                m = plsc.load_gather(tbl_v, [pv])
                cd = plsc.load_gather(tgt_v, [rv])
                ob = jnp.where(m != 0, cd, wv)
                obf_v[l, pl.ds(off, 16)] = ob
                mask_v[l, pl.ds(off, 16)] = m
                idx_v[pl.ds(l * BPW + off, 16)] = ob
            return _
        lax.fori_loop(c * LC, (c + 1) * LC, step, 0)

    def fire_gather(c):
        idx = idx_v.at[pl.ds(c * CROWS, CROWS)]
        return pltpu.async_copy(lut_h.at[idx], rows[c % 2], gsem[c % 2])

    def transpose_chunk(c):
        rv = rows[c % 2]
        tv = trans[c % 2]

        def tstep(lc, _):
            for ch in range(CH):
                cv = jnp.full((16,), ch, jnp.int32)
                for j in range(BPW // 16):
                    ridx = iota + (lc * BPW + j * 16)
                    v = plsc.load_gather(rv, [ridx, cv])
                    tv[lc, ch, pl.ds(j * 16, 16)] = v
            return _
        lax.fori_loop(0, LC, tstep, 0)

    def fire_out(c):
        return pltpu.async_copy(
            trans[c % 2], oc_h.at[pl.ds(c * LC, LC), :, pl.ds(b0, BPW)],
            osem[c % 2])

    gh = [None] * NCHUNK
    oh = [None] * NCHUNK
    for c in range(NCHUNK):
        compute_chunk(c)
        gh[c] = fire_gather(c)
        if c >= 1:
            gh[c - 1].wait()
            if c - 1 >= 2:
                oh[c - 3].wait()
            transpose_chunk(c - 1)
            oh[c - 1] = fire_out(c - 1)

    hw = [
        pltpu.async_copy(obf_v, ow_h.at[:, pl.ds(b0, BPW)], wsem),
        pltpu.async_copy(mask_v, om_h.at[:, pl.ds(b0, BPW)], wsem),
    ]

    c = NCHUNK - 1
    gh[c].wait()
    oh[c - 2].wait()
    transpose_chunk(c)
    oh[c] = fire_out(c)
    oh[c - 1].wait()
    oh[c].wait()
    for h in hw:
        h.wait()


_mesh = plsc.VectorSubcoreMesh(core_axis_name="c", subcore_axis_name="s")

_sc_call = functools.partial(
    pl.kernel,
    mesh=_mesh,
    compiler_params=pltpu.CompilerParams(
        needs_layout_passes=False, use_tc_tiling_on_sc=False),
    out_type=(
        jax.ShapeDtypeStruct((L, B), jnp.int32),
        jax.ShapeDtypeStruct((L, CH, B), jnp.float32),
        jax.ShapeDtypeStruct((L, B), jnp.int32),
    ),
    scratch_types=[
        pltpu.VMEM((16,), jnp.int32),
        pltpu.VMEM((N_TGT,), jnp.int32),
        pltpu.VMEM((TBL,), jnp.int32),
        pltpu.VMEM((L, BPW), jnp.int32),
        pltpu.VMEM((L, BPW), jnp.int32),
        pltpu.VMEM((L, BPW), jnp.int32),
        pltpu.VMEM((L, BPW), jnp.int32),
        pltpu.VMEM((L, BPW), jnp.int32),
        pltpu.VMEM((PW,), jnp.int32),
        pltpu.VMEM((CROWS, CH), jnp.float32),
        pltpu.VMEM((CROWS, CH), jnp.float32),
        pltpu.VMEM((LC, CH, BPW), jnp.float32),
        pltpu.VMEM((LC, CH, BPW), jnp.float32),
        pltpu.SemaphoreType.DMA,
        pltpu.SemaphoreType.DMA,
        pltpu.SemaphoreType.DMA,
        pltpu.SemaphoreType.DMA,
        pltpu.SemaphoreType.DMA,
        pltpu.SemaphoreType.DMA,
    ],
)(_body)


@jax.jit
def kernel(word, char, pos, lut, tgtwords, rand_idx, xpos_ids):
    pad = jnp.arange(48, 48 + 16 - xpos_ids.shape[0], dtype=jnp.int32)
    xp = jnp.concatenate([xpos_ids, pad])
    owt, oct_, omt = _sc_call(word.T, pos.T, rand_idx.T, lut, tgtwords, xp)
    return (
        owt.T,
        jnp.transpose(oct_, (2, 0, 1)).astype(char.dtype),
        pos,
        (omt != 0).T,
    )


# speedup vs baseline: 7.9763x; 1.3624x over previous
import functools

import jax
import jax.numpy as jnp
from jax import lax
from jax.experimental import pallas as pl
from jax.experimental.pallas import tpu as pltpu
from jax.experimental.pallas import tpu_sc as plsc

B = 4096
L = 50
CH = 16
NW = 32
BPW = B // NW
PW = BPW * L
N_TGT = 32
TBL = 64
LC = 10
NCHUNK = L // LC
CROWS = LC * BPW


def _body(word_h, pos_h, rand_h, lut_h, tgt_h, xp_h,
          ow_h, oc_h, om_h,
          xp_v, tgt_v, tbl_v, word_v, pos_v, rand_v, obf_v, mask_v,
          idx_v, rows0_v, rows1_v, trans0_v, trans1_v,
          insem, wsem, gsem0, gsem1, osem0, osem1):
    wid = lax.axis_index("s") * 2 + lax.axis_index("c")
    b0 = wid * BPW

    rows = (rows0_v, rows1_v)
    trans = (trans0_v, trans1_v)
    gsem = (gsem0, gsem1)
    osem = (osem0, osem1)

    h_in = [
        pltpu.async_copy(word_h.at[:, pl.ds(b0, BPW)], word_v, insem),
        pltpu.async_copy(pos_h.at[:, pl.ds(b0, BPW)], pos_v, insem),
        pltpu.async_copy(rand_h.at[:, pl.ds(b0, BPW)], rand_v, insem),
    ]
    pltpu.sync_copy(xp_h, xp_v)
    pltpu.sync_copy(tgt_h, tgt_v)

    zeros = jnp.zeros((16,), jnp.int32)
    for i in range(TBL // 16):
        tbl_v[pl.ds(i * 16, 16)] = zeros
    xv = jnp.clip(xp_v[...], 0, TBL - 1)
    plsc.store_scatter(tbl_v, [xv], jnp.ones((16,), jnp.int32))

    for h in h_in:
        h.wait()

    iota = lax.iota(jnp.int32, 16)

    def compute_chunk(c):
        @plsc.parallel_loop(c * LC, (c + 1) * LC, unroll=2)
        def step(l):
            for j in range(BPW // 16):
                off = j * 16
                pv = pos_v[l, pl.ds(off, 16)]
                rv = rand_v[l, pl.ds(off, 16)]
                wv = word_v[l, pl.ds(off, 16)]
                m = plsc.load_gather(tbl_v, [pv])
                cd = plsc.load_gather(tgt_v, [rv])
                ob = jnp.where(m != 0, cd, wv)
                obf_v[l, pl.ds(off, 16)] = ob
                mask_v[l, pl.ds(off, 16)] = m
                idx_v[pl.ds(l * BPW + off, 16)] = ob

    def fire_gather(c):
        idx = idx_v.at[pl.ds(c * CROWS, CROWS)]
        return pltpu.async_copy(lut_h.at[idx], rows[c % 2], gsem[c % 2])

    def transpose_chunk(c):
        rv = rows[c % 2]
        tv = trans[c % 2]

        @plsc.parallel_loop(0, LC * CH, unroll=2)
        def tstep(t):
            lc = t // CH
            ch = t % CH
            cv = jnp.full((16,), 0, jnp.int32) + ch
            for j in range(BPW // 16):
                ridx = iota + (lc * BPW + j * 16)
                v = plsc.load_gather(rv, [ridx, cv])
                tv[lc, ch, pl.ds(j * 16, 16)] = v

    def fire_out(c):
        return pltpu.async_copy(
            trans[c % 2], oc_h.at[pl.ds(c * LC, LC), :, pl.ds(b0, BPW)],
            osem[c % 2])

    gh = [None] * NCHUNK
    oh = [None] * NCHUNK
    for c in range(NCHUNK):
        compute_chunk(c)
        gh[c] = fire_gather(c)
        if c >= 1:
            gh[c - 1].wait()
            if c - 1 >= 2:
                oh[c - 3].wait()
            transpose_chunk(c - 1)
            oh[c - 1] = fire_out(c - 1)

    hw = [
        pltpu.async_copy(obf_v, ow_h.at[:, pl.ds(b0, BPW)], wsem),
        pltpu.async_copy(mask_v, om_h.at[:, pl.ds(b0, BPW)], wsem),
    ]

    c = NCHUNK - 1
    gh[c].wait()
    oh[c - 2].wait()
    transpose_chunk(c)
    oh[c] = fire_out(c)
    oh[c - 1].wait()
    oh[c].wait()
    for h in hw:
        h.wait()


_mesh = plsc.VectorSubcoreMesh(core_axis_name="c", subcore_axis_name="s")

_sc_call = functools.partial(
    pl.kernel,
    mesh=_mesh,
    compiler_params=pltpu.CompilerParams(
        needs_layout_passes=False, use_tc_tiling_on_sc=False),
    out_type=(
        jax.ShapeDtypeStruct((L, B), jnp.int32),
        jax.ShapeDtypeStruct((L, CH, B), jnp.float32),
        jax.ShapeDtypeStruct((L, B), jnp.int32),
    ),
    scratch_types=[
        pltpu.VMEM((16,), jnp.int32),
        pltpu.VMEM((N_TGT,), jnp.int32),
        pltpu.VMEM((TBL,), jnp.int32),
        pltpu.VMEM((L, BPW), jnp.int32),
        pltpu.VMEM((L, BPW), jnp.int32),
        pltpu.VMEM((L, BPW), jnp.int32),
        pltpu.VMEM((L, BPW), jnp.int32),
        pltpu.VMEM((L, BPW), jnp.int32),
        pltpu.VMEM((PW,), jnp.int32),
        pltpu.VMEM((CROWS, CH), jnp.float32),
        pltpu.VMEM((CROWS, CH), jnp.float32),
        pltpu.VMEM((LC, CH, BPW), jnp.float32),
        pltpu.VMEM((LC, CH, BPW), jnp.float32),
        pltpu.SemaphoreType.DMA,
        pltpu.SemaphoreType.DMA,
        pltpu.SemaphoreType.DMA,
        pltpu.SemaphoreType.DMA,
        pltpu.SemaphoreType.DMA,
        pltpu.SemaphoreType.DMA,
    ],
)(_body)


@jax.jit
def kernel(word, char, pos, lut, tgtwords, rand_idx, xpos_ids):
    pad = jnp.arange(48, 48 + 16 - xpos_ids.shape[0], dtype=jnp.int32)
    xp = jnp.concatenate([xpos_ids, pad])
    owt, oct_, omt = _sc_call(word.T, pos.T, rand_idx.T, lut, tgtwords, xp)
    return (
        owt.T,
        jnp.transpose(oct_, (2, 0, 1)).astype(char.dtype),
        pos,
        (omt != 0).T,
    )


# speedup vs baseline: 7.9817x; 1.0007x over previous
import functools

import jax
import jax.numpy as jnp
from jax import lax
from jax.experimental import pallas as pl
from jax.experimental.pallas import tpu as pltpu
from jax.experimental.pallas import tpu_sc as plsc

B = 4096
L = 50
CH = 16
NW = 32
BPW = B // NW
PW = BPW * L
N_TGT = 32
TBL = 64
LC = 10
NCHUNK = L // LC
CROWS = LC * BPW


def _body(word_h, pos_h, rand_h, lut_h, tgt_h, xp_h,
          ow_h, oc_h, om_h,
          xp_v, tgt_v, tbl_v, word_v, pos_v, rand_v, obf_v, mask_v,
          idx_v, rows0_v, rows1_v, trans0_v, trans1_v,
          insem, wsem, gsem0, gsem1, osem0, osem1):
    wid = lax.axis_index("s") * 2 + lax.axis_index("c")
    b0 = wid * BPW

    rows = (rows0_v, rows1_v)
    trans = (trans0_v, trans1_v)
    gsem = (gsem0, gsem1)
    osem = (osem0, osem1)

    h_in = [
        pltpu.async_copy(word_h.at[:, pl.ds(b0, BPW)], word_v, insem),
        pltpu.async_copy(pos_h.at[:, pl.ds(b0, BPW)], pos_v, insem),
        pltpu.async_copy(rand_h.at[:, pl.ds(b0, BPW)], rand_v, insem),
    ]
    pltpu.sync_copy(xp_h, xp_v)
    pltpu.sync_copy(tgt_h, tgt_v)

    zeros = jnp.zeros((16,), jnp.int32)
    for i in range(TBL // 16):
        tbl_v[pl.ds(i * 16, 16)] = zeros
    xv = jnp.clip(xp_v[...], 0, TBL - 1)
    plsc.store_scatter(tbl_v, [xv], jnp.ones((16,), jnp.int32))

    for h in h_in:
        h.wait()

    iota = lax.iota(jnp.int32, 16)

    def compute_chunk(c):
        @plsc.parallel_loop(c * LC, (c + 1) * LC, unroll=4)
        def step(l):
            for j in range(BPW // 16):
                off = j * 16
                pv = pos_v[l, pl.ds(off, 16)]
                rv = rand_v[l, pl.ds(off, 16)]
                wv = word_v[l, pl.ds(off, 16)]
                m = plsc.load_gather(tbl_v, [pv])
                cd = plsc.load_gather(tgt_v, [rv])
                ob = jnp.where(m != 0, cd, wv)
                obf_v[l, pl.ds(off, 16)] = ob
                mask_v[l, pl.ds(off, 16)] = m
                idx_v[pl.ds(l * BPW + off, 16)] = ob

    def fire_gather(c):
        idx = idx_v.at[pl.ds(c * CROWS, CROWS)]
        return pltpu.async_copy(lut_h.at[idx], rows[c % 2], gsem[c % 2])

    def transpose_chunk(c):
        rv = rows[c % 2]
        tv = trans[c % 2]

        @plsc.parallel_loop(0, LC * CH, unroll=4)
        def tstep(t):
            lc = t // CH
            ch = t % CH
            cv = jnp.full((16,), 0, jnp.int32) + ch
            for j in range(BPW // 16):
                ridx = iota + (lc * BPW + j * 16)
                v = plsc.load_gather(rv, [ridx, cv])
                tv[lc, ch, pl.ds(j * 16, 16)] = v

    def fire_out(c):
        return pltpu.async_copy(
            trans[c % 2], oc_h.at[pl.ds(c * LC, LC), :, pl.ds(b0, BPW)],
            osem[c % 2])

    gh = [None] * NCHUNK
    oh = [None] * NCHUNK
    for c in range(NCHUNK):
        compute_chunk(c)
        gh[c] = fire_gather(c)
        if c >= 1:
            gh[c - 1].wait()
            if c - 1 >= 2:
                oh[c - 3].wait()
            transpose_chunk(c - 1)
            oh[c - 1] = fire_out(c - 1)

    hw = [
        pltpu.async_copy(obf_v, ow_h.at[:, pl.ds(b0, BPW)], wsem),
        pltpu.async_copy(mask_v, om_h.at[:, pl.ds(b0, BPW)], wsem),
    ]

    c = NCHUNK - 1
    gh[c].wait()
    oh[c - 2].wait()
    transpose_chunk(c)
    oh[c] = fire_out(c)
    oh[c - 1].wait()
    oh[c].wait()
    for h in hw:
        h.wait()


_mesh = plsc.VectorSubcoreMesh(core_axis_name="c", subcore_axis_name="s")

_sc_call = functools.partial(
    pl.kernel,
    mesh=_mesh,
    compiler_params=pltpu.CompilerParams(
        needs_layout_passes=False, use_tc_tiling_on_sc=False),
    out_type=(
        jax.ShapeDtypeStruct((L, B), jnp.int32),
        jax.ShapeDtypeStruct((L, CH, B), jnp.float32),
        jax.ShapeDtypeStruct((L, B), jnp.int32),
    ),
    scratch_types=[
        pltpu.VMEM((16,), jnp.int32),
        pltpu.VMEM((N_TGT,), jnp.int32),
        pltpu.VMEM((TBL,), jnp.int32),
        pltpu.VMEM((L, BPW), jnp.int32),
        pltpu.VMEM((L, BPW), jnp.int32),
        pltpu.VMEM((L, BPW), jnp.int32),
        pltpu.VMEM((L, BPW), jnp.int32),
        pltpu.VMEM((L, BPW), jnp.int32),
        pltpu.VMEM((PW,), jnp.int32),
        pltpu.VMEM((CROWS, CH), jnp.float32),
        pltpu.VMEM((CROWS, CH), jnp.float32),
        pltpu.VMEM((LC, CH, BPW), jnp.float32),
        pltpu.VMEM((LC, CH, BPW), jnp.float32),
        pltpu.SemaphoreType.DMA,
        pltpu.SemaphoreType.DMA,
        pltpu.SemaphoreType.DMA,
        pltpu.SemaphoreType.DMA,
        pltpu.SemaphoreType.DMA,
        pltpu.SemaphoreType.DMA,
    ],
)(_body)


@jax.jit
def kernel(word, char, pos, lut, tgtwords, rand_idx, xpos_ids):
    pad = jnp.arange(48, 48 + 16 - xpos_ids.shape[0], dtype=jnp.int32)
    xp = jnp.concatenate([xpos_ids, pad])
    owt, oct_, omt = _sc_call(word.T, pos.T, rand_idx.T, lut, tgtwords, xp)
    return (
        owt.T,
        jnp.transpose(oct_, (2, 0, 1)).astype(char.dtype),
        pos,
        (omt != 0).T,
    )
